# baseline (speedup 1.0000x reference)
import jax
import jax.numpy as jnp
from jax import lax
from jax.experimental import pallas as pl
from jax.experimental.pallas import tpu as pltpu
from jax.experimental.pallas import tpu_sc as plsc

NUM_EMBEDDINGS = 100000
EMBEDDING_DIM = 128
BATCH = 4096
SEQ = 200

_NC = 2
_NS = 16
_NW = _NC * _NS

_B = BATCH * SEQ
_G = 128

_STEPS = 160
_ROWS_PER_W = _STEPS * _G
_B_SC = _ROWS_PER_W * _NW
_B_TC = _B - _B_SC

_NRB = 5
_GP = 3
_NIB = 10
_T = _STEPS // _NIB

_R = 512
_NBLK = _B_TC // _R


def _sc_body(idx_hbm, table_hbm, out_hbm, ibufs, rbufs, gsem, osem, isem):
    wid = lax.axis_index("s") * _NC + lax.axis_index("c")
    idx_row0 = wid * _STEPS
    row_base = wid * _ROWS_PER_W

    def start_idx(s, k):
        pltpu.async_copy(idx_hbm.at[idx_row0 + s], ibufs.at[k], isem)

    def wait_idx(k):
        pltpu.make_async_copy(idx_hbm.at[idx_row0], ibufs.at[k], isem).wait()

    def start_gather(j, k):
        pltpu.async_copy(table_hbm.at[ibufs.at[k]], rbufs.at[j], gsem)

    def wait_gather(j, k):
        pltpu.make_async_copy(table_hbm.at[ibufs.at[k]], rbufs.at[j], gsem).wait()

    def start_scatter(s, j):
        pltpu.async_copy(rbufs.at[j], out_hbm.at[pl.ds(row_base + s * _G, _G)], osem)

    def wait_scatter(j):
        pltpu.make_async_copy(
            rbufs.at[j], out_hbm.at[pl.ds(row_base, _G)], osem
        ).wait()

    def step(s, jI, first_pass):
        j = jI % _NRB
        wait_gather(j, jI)
        start_scatter(s, j)
        kg = (jI + _GP) % _NIB
        wait_idx(kg)
        if not (first_pass and jI < 2):
            wait_scatter((j + _GP) % _NRB)
        start_gather((j + _GP) % _NRB, kg)
        start_idx(s + 6, (jI + 6) % _NIB)

    for k in range(6):
        start_idx(k, k)
    for k in range(_GP):
        wait_idx(k)
        start_gather(k, k)

    for jI in range(_NIB):
        step(jI, jI, True)

    def pass_body(t, carry):
        s0 = t * _NIB
        for jI in range(_NIB):
            step(s0 + jI, jI, False)
        return carry

    lax.fori_loop(1, _T - 1, pass_body, 0)

    s0 = _STEPS - _NIB
    for jI in range(_NIB):
        s = s0 + jI
        j = jI % _NRB
        wait_gather(j, jI)
        start_scatter(s, j)
        if jI < _NIB - _GP:
            kg = (jI + _GP) % _NIB
            wait_idx(kg)
            wait_scatter((j + _GP) % _NRB)
            start_gather((j + _GP) % _NRB, kg)
        if jI < 4:
            start_idx(s + 6, (jI + 6) % _NIB)

    for _ in range(_NRB):
        wait_scatter(0)


def _tc_body(idx_ref, table_ref, out_ref):
    def row(r, carry):
        out_ref[pl.ds(r, 1), :] = table_ref[pl.ds(idx_ref[0, 0, r], 1), :]
        return carry

    lax.fori_loop(0, _R, row, 0, unroll=8)


@jax.jit
def kernel(x, param):
    flat = x.reshape(_B).astype(jnp.int32)
    idx_sc = flat[:_B_SC].reshape(_B_SC // _G, _G)
    idx_tc = flat[_B_SC:].reshape(_NBLK, 1, _R)

    out_tc = pl.pallas_call(
        _tc_body,
        grid=(_NBLK,),
        in_specs=[
            pl.BlockSpec((1, 1, _R), lambda i: (i, 0, 0), memory_space=pltpu.SMEM),
            pl.BlockSpec((NUM_EMBEDDINGS, EMBEDDING_DIM), lambda i: (0, 0)),
        ],
        out_specs=pl.BlockSpec((_R, EMBEDDING_DIM), lambda i: (i, 0)),
        out_shape=jax.ShapeDtypeStruct((_B_TC, EMBEDDING_DIM), jnp.float32),
    )(idx_tc, param)

    mesh = plsc.VectorSubcoreMesh(core_axis_name="c", subcore_axis_name="s")
    out_sc = pl.kernel(
        _sc_body,
        out_type=jax.ShapeDtypeStruct((_B, EMBEDDING_DIM), jnp.float32),
        mesh=mesh,
        scratch_types=[
            pltpu.VMEM((_NIB, _G), jnp.int32),
            pltpu.VMEM((_NRB, _G, EMBEDDING_DIM), jnp.float32),
            pltpu.SemaphoreType.DMA,
            pltpu.SemaphoreType.DMA,
            pltpu.SemaphoreType.DMA,
        ],
    )(idx_sc, param)

    out = lax.dynamic_update_slice(out_sc, out_tc, (_B_SC, 0))
    return out.reshape(BATCH, SEQ, EMBEDDING_DIM)


# speedup vs baseline: 1.4290x; 1.4290x over previous
import jax
import jax.numpy as jnp
from jax import lax
from jax.experimental import pallas as pl
from jax.experimental.pallas import tpu as pltpu
from jax.experimental.pallas import tpu_sc as plsc

NUM_EMBEDDINGS = 100000
EMBEDDING_DIM = 128
BATCH = 4096
SEQ = 200

_NC = 2
_NS = 16
_NW = _NC * _NS

_B = BATCH * SEQ
_G = 128
_ROWS_PER_W = _B // _NW
_STEPS = _ROWS_PER_W // _G
_NRB = 5
_GP = 3
_NIB = 10
_T = _STEPS // _NIB


def _body(idx_hbm, table_hbm, out_hbm, ibufs, rbufs, gsem, osem, isem):
    wid = lax.axis_index("s") * _NC + lax.axis_index("c")
    idx_row0 = wid * _STEPS
    row_base = wid * _ROWS_PER_W

    def start_idx(s, k):
        pltpu.async_copy(idx_hbm.at[idx_row0 + s], ibufs.at[k], isem)

    def wait_idx(k):
        pltpu.make_async_copy(idx_hbm.at[idx_row0], ibufs.at[k], isem).wait()

    def start_gather(j, k):
        pltpu.async_copy(table_hbm.at[ibufs.at[k]], rbufs.at[j], gsem)

    def wait_gather(j, k):
        pltpu.make_async_copy(table_hbm.at[ibufs.at[k]], rbufs.at[j], gsem).wait()

    def start_scatter(s, j):
        pltpu.async_copy(rbufs.at[j], out_hbm.at[pl.ds(row_base + s * _G, _G)], osem)

    def wait_scatter(j):
        pltpu.make_async_copy(
            rbufs.at[j], out_hbm.at[pl.ds(row_base, _G)], osem
        ).wait()

    def step(s, jI, first_pass):
        j = jI % _NRB
        wait_gather(j, jI)
        start_scatter(s, j)
        kg = (jI + _GP) % _NIB
        wait_idx(kg)
        if not (first_pass and jI < 2):
            wait_scatter((j + _GP) % _NRB)
        start_gather((j + _GP) % _NRB, kg)
        start_idx(s + 6, (jI + 6) % _NIB)

    for k in range(6):
        start_idx(k, k)
    for k in range(_GP):
        wait_idx(k)
        start_gather(k, k)

    for jI in range(_NIB):
        step(jI, jI, True)

    def pass_body(t, carry):
        s0 = t * _NIB
        for jI in range(_NIB):
            step(s0 + jI, jI, False)
        return carry

    lax.fori_loop(1, _T - 1, pass_body, 0)

    s0 = _STEPS - _NIB
    for jI in range(_NIB):
        s = s0 + jI
        j = jI % _NRB
        wait_gather(j, jI)
        start_scatter(s, j)
        if jI < _NIB - _GP:
            kg = (jI + _GP) % _NIB
            wait_idx(kg)
            wait_scatter((j + _GP) % _NRB)
            start_gather((j + _GP) % _NRB, kg)
        if jI < 4:
            start_idx(s + 6, (jI + 6) % _NIB)

    for _ in range(_NRB):
        wait_scatter(0)


@jax.jit
def kernel(x, param):
    idx = x.reshape(_B // _G, _G).astype(jnp.int32)
    mesh = plsc.VectorSubcoreMesh(core_axis_name="c", subcore_axis_name="s")
    out = pl.kernel(
        _body,
        out_type=jax.ShapeDtypeStruct((_B, EMBEDDING_DIM), jnp.float32),
        mesh=mesh,
        scratch_types=[
            pltpu.VMEM((_NIB, _G), jnp.int32),
            pltpu.VMEM((_NRB, _G, EMBEDDING_DIM), jnp.float32),
            pltpu.SemaphoreType.DMA,
            pltpu.SemaphoreType.DMA,
            pltpu.SemaphoreType.DMA,
        ],
    )(idx, param)
    return out.reshape(BATCH, SEQ, EMBEDDING_DIM)
